# baseline (speedup 1.0000x reference)
import functools

import jax
import jax.numpy as jnp
from jax.experimental import pallas as pl
from jax.experimental.pallas import tpu as pltpu

DIM = 1024
E = 8
K = 2
TS = 1024


def _router_kernel(x0_ref, wr_ref, br_ref, sel_ref):
    scores = jax.lax.dot_general(
        x0_ref[...], wr_ref[...], (((1,), (0,)), ((), ())),
        preferred_element_type=jnp.float32,
        precision=jax.lax.Precision.HIGHEST,
    ) + br_ref[...]
    neg = jnp.finfo(jnp.float32).min
    i0 = jnp.argmax(scores, axis=1, keepdims=True)
    col = jax.lax.broadcasted_iota(jnp.int32, scores.shape, 1)
    masked = jnp.where(col == i0, neg, scores)
    i1 = jnp.argmax(masked, axis=1, keepdims=True)
    sel_ref[...] = jnp.concatenate([i0, i1], axis=1).astype(jnp.int32)


def _gelu_exact(h):
    return 0.5 * h * (1.0 + jax.lax.erf(h * 0.7071067811865476))


def _mlp_kernel(sel_ref, x_ref, w1a_ref, w1b_ref, w2a_ref, w2b_ref,
                b1a_ref, b1b_ref, b2a_ref, b2b_ref, o_ref):
    del sel_ref
    dot = functools.partial(jnp.dot, preferred_element_type=jnp.float32,
                            precision=jax.lax.Precision.DEFAULT)
    x_blk = x_ref[0]
    h0 = _gelu_exact(dot(x_blk, w1a_ref[0]) + b1a_ref[0])
    y0 = dot(h0, w2a_ref[0])
    h1 = _gelu_exact(dot(x_blk, w1b_ref[0]) + b1b_ref[0])
    y1 = dot(h1, w2b_ref[0])
    o_ref[0] = 0.5 * (y0 + y1 + b2a_ref[0] + b2b_ref[0])


@jax.jit
def kernel(x, Wr, br, W1, b1, W2, b2):
    B, S, _ = x.shape

    sel = pl.pallas_call(
        _router_kernel,
        grid=(1,),
        in_specs=[
            pl.BlockSpec((B, DIM), lambda i: (0, 0)),
            pl.BlockSpec((DIM, E), lambda i: (0, 0)),
            pl.BlockSpec((1, E), lambda i: (0, 0)),
        ],
        out_specs=pl.BlockSpec((B, K), lambda i: (0, 0)),
        out_shape=jax.ShapeDtypeStruct((B, K), jnp.int32),
    )(x.reshape(B, S * DIM), Wr, br.reshape(1, E))

    def wspec(k):
        return pl.BlockSpec((1, DIM, DIM), lambda b, s, sel: (sel[b, k], 0, 0))

    def bspec(k):
        return pl.BlockSpec((1, 1, DIM), lambda b, s, sel: (sel[b, k], 0, 0))

    grid_spec = pltpu.PrefetchScalarGridSpec(
        num_scalar_prefetch=1,
        grid=(B, S // TS),
        in_specs=[
            pl.BlockSpec((1, TS, DIM), lambda b, s, sel: (b, s, 0)),
            wspec(0), wspec(1), wspec(0), wspec(1),
            bspec(0), bspec(1), bspec(0), bspec(1),
        ],
        out_specs=pl.BlockSpec((1, TS, DIM), lambda b, s, sel: (b, s, 0)),
    )

    out = pl.pallas_call(
        _mlp_kernel,
        grid_spec=grid_spec,
        out_shape=jax.ShapeDtypeStruct((B, S, DIM), jnp.float32),
        compiler_params=pltpu.CompilerParams(
            dimension_semantics=("parallel", "arbitrary")),
    )(sel, x, W1, W1, W2, W2,
      b1.reshape(E, 1, DIM), b1.reshape(E, 1, DIM),
      b2.reshape(E, 1, DIM), b2.reshape(E, 1, DIM))
    return out


# speedup vs baseline: 1.5099x; 1.5099x over previous
import functools

import jax
import jax.numpy as jnp
from jax.experimental import pallas as pl
from jax.experimental.pallas import tpu as pltpu

DIM = 1024
E = 8
K = 2
TS = 1024


def _router_kernel(x0_ref, wr_ref, br_ref, sel_ref):
    scores = jax.lax.dot_general(
        x0_ref[:, 0, :], wr_ref[...], (((1,), (0,)), ((), ())),
        preferred_element_type=jnp.float32,
        precision=jax.lax.Precision.HIGHEST,
    ) + br_ref[...]
    neg = jnp.finfo(jnp.float32).min
    i0 = jnp.argmax(scores, axis=1, keepdims=True)
    col = jax.lax.broadcasted_iota(jnp.int32, scores.shape, 1)
    masked = jnp.where(col == i0, neg, scores)
    i1 = jnp.argmax(masked, axis=1, keepdims=True)
    sel_ref[...] = jnp.concatenate([i0, i1], axis=1).astype(jnp.int32)


def _gelu_exact(h):
    return 0.5 * h * (1.0 + jax.lax.erf(h * 0.7071067811865476))


def _mlp_kernel(sel_ref, x_ref, w1a_ref, w1b_ref, w2a_ref, w2b_ref,
                b1a_ref, b1b_ref, b2a_ref, b2b_ref, o_ref):
    del sel_ref
    dot = functools.partial(jnp.dot, preferred_element_type=jnp.float32,
                            precision=jax.lax.Precision.DEFAULT)
    x_blk = x_ref[0]
    h0 = _gelu_exact(dot(x_blk, w1a_ref[0]) + b1a_ref[0])
    y0 = dot(h0, w2a_ref[0])
    h1 = _gelu_exact(dot(x_blk, w1b_ref[0]) + b1b_ref[0])
    y1 = dot(h1, w2b_ref[0])
    o_ref[0] = 0.5 * (y0 + y1 + b2a_ref[0] + b2b_ref[0])


@jax.jit
def kernel(x, Wr, br, W1, b1, W2, b2):
    B, S, _ = x.shape

    sel = pl.pallas_call(
        _router_kernel,
        grid=(1,),
        in_specs=[
            pl.BlockSpec((B, 8, DIM), lambda i: (0, 0, 0)),
            pl.BlockSpec((DIM, E), lambda i: (0, 0)),
            pl.BlockSpec((1, E), lambda i: (0, 0)),
        ],
        out_specs=pl.BlockSpec((B, K), lambda i: (0, 0)),
        out_shape=jax.ShapeDtypeStruct((B, K), jnp.int32),
    )(x, Wr, br.reshape(1, E))

    def wspec(k):
        return pl.BlockSpec((1, DIM, DIM), lambda b, s, sel: (sel[b, k], 0, 0))

    def bspec(k):
        return pl.BlockSpec((1, 1, DIM), lambda b, s, sel: (sel[b, k], 0, 0))

    grid_spec = pltpu.PrefetchScalarGridSpec(
        num_scalar_prefetch=1,
        grid=(B, S // TS),
        in_specs=[
            pl.BlockSpec((1, TS, DIM), lambda b, s, sel: (b, s, 0)),
            wspec(0), wspec(1), wspec(0), wspec(1),
            bspec(0), bspec(1), bspec(0), bspec(1),
        ],
        out_specs=pl.BlockSpec((1, TS, DIM), lambda b, s, sel: (b, s, 0)),
    )

    out = pl.pallas_call(
        _mlp_kernel,
        grid_spec=grid_spec,
        out_shape=jax.ShapeDtypeStruct((B, S, DIM), jnp.float32),
        compiler_params=pltpu.CompilerParams(
            dimension_semantics=("parallel", "arbitrary")),
    )(sel, x, W1, W1, W2, W2,
      b1.reshape(E, 1, DIM), b1.reshape(E, 1, DIM),
      b2.reshape(E, 1, DIM), b2.reshape(E, 1, DIM))
    return out
